# baseline (speedup 1.0000x reference)
import functools

import jax
import jax.numpy as jnp
from jax import lax
from jax.experimental import pallas as pl
from jax.experimental.pallas import tpu as pltpu
from jax.experimental.pallas import tpu_sc as plsc

NC, NS = 2, 16
NW = NC * NS
BLK = 128


def _router_body(x_ref, wg_ref, bg_ref, top_ref, slot_ref, be_ref, nbu_ref,
                 xbf_ref):
    xbf_ref[...] = x_ref[...].astype(jnp.bfloat16)
    logits = lax.dot_general(x_ref[...], wg_ref[...], (((1,), (1,)), ((), ())),
                             preferred_element_type=jnp.float32)
    logits = logits + bg_ref[...]
    t, e = logits.shape
    nb = t // BLK + e
    maxv = jnp.max(logits, axis=1, keepdims=True)
    ids = lax.broadcasted_iota(jnp.int32, logits.shape, 1)
    picked = jnp.where(logits == maxv, ids, e)
    top = jnp.min(picked, axis=1, keepdims=True)
    top_ref[...] = top

    ohb = ids == top
    ohf = ohb.astype(jnp.bfloat16)
    r_io = lax.broadcasted_iota(jnp.int32, (t, t), 0)
    c_io = lax.broadcasted_iota(jnp.int32, (t, t), 1)
    tril = (r_io >= c_io).astype(jnp.bfloat16)
    csum = lax.dot_general(tril, ohf, (((1,), (0,)), ((), ())),
                           preferred_element_type=jnp.float32)
    rank = jnp.sum(jnp.where(ohb, csum, 0.0), axis=1, keepdims=True) - 1.0
    counts_i = csum[t - 1:t, :].astype(jnp.int32)
    nblk_i = lax.shift_right_logical(counts_i + (BLK - 1), BLK.bit_length() - 1)
    e_r = lax.broadcasted_iota(jnp.int32, (e, e), 0)
    e_c = lax.broadcasted_iota(jnp.int32, (e, e), 1)
    tril_e = (e_r <= e_c).astype(jnp.bfloat16)
    blk_endf = lax.dot_general(nblk_i.astype(jnp.bfloat16), tril_e,
                               (((1,), (0,)), ((), ())),
                               preferred_element_type=jnp.float32)
    blk_end_i = blk_endf.astype(jnp.int32)
    start_tokf = (blk_endf - nblk_i.astype(jnp.float32)) * float(BLK)
    slot_basef = jnp.sum(jnp.where(ohb, start_tokf, 0.0), axis=1, keepdims=True)
    slot_ref[...] = (slot_basef + rank).astype(jnp.int32)
    nbu = blk_end_i[:, e - 1:e]
    nbu_ref[...] = nbu
    b_io = lax.broadcasted_iota(jnp.int32, (nb, e), 0)
    be_raw = jnp.sum((blk_end_i <= b_io).astype(jnp.int32),
                     axis=1, keepdims=True)
    last_e = jnp.sum((blk_end_i <= nbu - 1).astype(jnp.int32),
                     axis=1, keepdims=True)
    be_ref[...] = jnp.where(b_io[:, 0:1] < nbu, be_raw, last_e)


def _router(x, Wg, bg):
    t, d = x.shape
    e = Wg.shape[0]
    nb = t // BLK + e
    return pl.pallas_call(
        _router_body,
        out_shape=(
            jax.ShapeDtypeStruct((t, 1), jnp.int32),
            jax.ShapeDtypeStruct((t, 1), jnp.int32),
            jax.ShapeDtypeStruct((nb, 1), jnp.int32),
            jax.ShapeDtypeStruct((1, 1), jnp.int32),
            jax.ShapeDtypeStruct((t, d), jnp.bfloat16),
        ),
    )(x, Wg, bg.reshape(1, -1))


def _ffn_body(be_ref, nblk_ref, slot_ref, xbf_ref, w1_ref, b1_ref, w2_ref,
              b2_ref, out_ref):
    @pl.when(pl.program_id(0) < nblk_ref[0])
    def _():
        t = slot_ref.shape[1]
        rio = (lax.broadcasted_iota(jnp.int32, (BLK, t), 0)
               + pl.program_id(0) * BLK)
        perm = (rio == slot_ref[...]).astype(jnp.bfloat16)
        xb = lax.dot_general(perm, xbf_ref[...], (((1,), (0,)), ((), ())),
                             preferred_element_type=jnp.float32
                             ).astype(jnp.bfloat16)
        w1 = w1_ref[0].astype(jnp.bfloat16)
        h = lax.dot_general(xb, w1, (((1,), (1,)), ((), ())),
                            preferred_element_type=jnp.float32)
        h = jnp.maximum(h + b1_ref[0], 0.0)
        w2 = w2_ref[0].astype(jnp.bfloat16)
        y = lax.dot_general(h.astype(jnp.bfloat16), w2, (((1,), (1,)), ((), ())),
                            preferred_element_type=jnp.float32)
        out_ref[...] = y + b2_ref[0]


def _grouped_ffn(block_expert, nblk_used, slot_row, x_bf, W1, b1, W2, b2):
    t, d = x_bf.shape
    e, h = b1.shape
    nb = t // BLK + e
    tp = nb * BLK
    grid_spec = pltpu.PrefetchScalarGridSpec(
        num_scalar_prefetch=2,
        grid=(nb,),
        in_specs=[
            pl.BlockSpec((1, t), lambda i, be, nu: (0, 0)),
            pl.BlockSpec((t, d), lambda i, be, nu: (0, 0)),
            pl.BlockSpec((1, h, d), lambda i, be, nu: (be[i], 0, 0)),
            pl.BlockSpec((1, 1, h), lambda i, be, nu: (be[i], 0, 0)),
            pl.BlockSpec((1, d, h), lambda i, be, nu: (be[i], 0, 0)),
            pl.BlockSpec((1, 1, d), lambda i, be, nu: (be[i], 0, 0)),
        ],
        out_specs=pl.BlockSpec((BLK, d), lambda i, be, nu: (i, 0)),
    )
    return pl.pallas_call(
        _ffn_body,
        grid_spec=grid_spec,
        out_shape=jax.ShapeDtypeStruct((tp, d), jnp.float32),
    )(block_expert, nblk_used, slot_row, x_bf, W1, b1.reshape(e, 1, h), W2,
      b2.reshape(e, 1, d))


def _sc_gather(table, idx):
    n = idx.shape[0]
    d = table.shape[1]
    rows_per_w = n // NW
    mesh = plsc.VectorSubcoreMesh(core_axis_name="c", subcore_axis_name="s")

    @functools.partial(
        pl.kernel,
        mesh=mesh,
        out_type=jax.ShapeDtypeStruct((n, d), jnp.float32),
        scratch_types=[
            pltpu.VMEM((rows_per_w,), jnp.int32),
            pltpu.VMEM((rows_per_w, d), jnp.float32),
            pltpu.SemaphoreType.DMA,
        ],
    )
    def k(table_hbm, idx_hbm, out_hbm, idx_v, rows_v, sem):
        wid = lax.axis_index("s") * NC + lax.axis_index("c")
        base = wid * rows_per_w
        pltpu.sync_copy(idx_hbm.at[pl.ds(base, rows_per_w)], idx_v)
        pltpu.async_copy(table_hbm.at[idx_v], rows_v, sem).wait()
        pltpu.sync_copy(rows_v, out_hbm.at[pl.ds(base, rows_per_w)])

    return k(table, idx)


def kernel(x, Wg, bg, W1, b1, W2, b2):
    t, d = x.shape
    e = Wg.shape[0]
    nb = t // BLK + e
    tp = nb * BLK

    top_i, slot, block_expert, nbu, x_bf = _router(x, Wg, bg)

    ys = _grouped_ffn(block_expert.reshape(nb), nbu.reshape(1),
                      slot.reshape(1, t), x_bf, W1, b1, W2, b2)
    out = _sc_gather(ys, slot.reshape(t))
    return (out, top_i)


# speedup vs baseline: 1.0883x; 1.0883x over previous
import functools

import jax
import jax.numpy as jnp
from jax import lax
from jax.experimental import pallas as pl
from jax.experimental.pallas import tpu as pltpu
from jax.experimental.pallas import tpu_sc as plsc

NC, NS = 2, 16
NW = NC * NS
BLK = 128


def _router_body(x_ref, wg_ref, bg_ref, top_ref, slot_ref, be_ref, nbu_ref,
                 xbf_ref):
    xbf_ref[...] = x_ref[...].astype(jnp.bfloat16)
    logits = lax.dot_general(wg_ref[...], x_ref[...], (((1,), (1,)), ((), ())),
                             preferred_element_type=jnp.float32)
    logits = logits + bg_ref[...]
    e, t = logits.shape
    nb = t // BLK + e
    maxv = jnp.max(logits, axis=0, keepdims=True)
    ids = lax.broadcasted_iota(jnp.int32, logits.shape, 0)
    picked = jnp.where(logits == maxv, ids, e)
    top = jnp.min(picked, axis=0, keepdims=True)
    top_ref[...] = top

    ohb = ids == top
    ohf = ohb.astype(jnp.bfloat16)
    r_io = lax.broadcasted_iota(jnp.int32, (t, t), 0)
    c_io = lax.broadcasted_iota(jnp.int32, (t, t), 1)
    triu = (r_io <= c_io).astype(jnp.bfloat16)
    csum = lax.dot_general(ohf, triu, (((1,), (0,)), ((), ())),
                           preferred_element_type=jnp.float32)
    rank = jnp.sum(jnp.where(ohb, csum, 0.0), axis=0, keepdims=True) - 1.0
    counts_i = csum[:, t - 1:t].astype(jnp.int32)
    nblk_i = lax.shift_right_logical(counts_i + (BLK - 1), BLK.bit_length() - 1)
    e_r = lax.broadcasted_iota(jnp.int32, (e, e), 0)
    e_c = lax.broadcasted_iota(jnp.int32, (e, e), 1)
    tril_e = (e_r >= e_c).astype(jnp.bfloat16)
    blk_endf = lax.dot_general(tril_e, nblk_i.astype(jnp.bfloat16),
                               (((1,), (0,)), ((), ())),
                               preferred_element_type=jnp.float32)
    blk_end_i = blk_endf.astype(jnp.int32)
    start_tokf = (blk_endf - nblk_i.astype(jnp.float32)) * float(BLK)
    slot_basef = jnp.sum(jnp.where(ohb, start_tokf, 0.0), axis=0, keepdims=True)
    slot_ref[...] = (slot_basef + rank).astype(jnp.int32)
    nbu = blk_end_i[e - 1:e, :]
    nbu_ref[...] = nbu
    b_io = lax.broadcasted_iota(jnp.int32, (e, nb), 1)
    be_raw = jnp.sum((blk_end_i <= b_io).astype(jnp.int32),
                     axis=0, keepdims=True)
    last_e = jnp.sum((blk_end_i <= nbu - 1).astype(jnp.int32),
                     axis=0, keepdims=True)
    be_ref[...] = jnp.where(b_io[0:1, :] < nbu, be_raw, last_e)


def _router(x, Wg, bg):
    t, d = x.shape
    e = Wg.shape[0]
    nb = t // BLK + e
    return pl.pallas_call(
        _router_body,
        out_shape=(
            jax.ShapeDtypeStruct((1, t), jnp.int32),
            jax.ShapeDtypeStruct((1, t), jnp.int32),
            jax.ShapeDtypeStruct((1, nb), jnp.int32),
            jax.ShapeDtypeStruct((1, 1), jnp.int32),
            jax.ShapeDtypeStruct((t, d), jnp.bfloat16),
        ),
    )(x, Wg, bg.reshape(-1, 1))


def _ffn_body(be_ref, nblk_ref, slot_ref, xbf_ref, w1_ref, b1_ref, w2_ref,
              b2_ref, out_ref):
    @pl.when(pl.program_id(0) < nblk_ref[0])
    def _():
        t = slot_ref.shape[1]
        rio = (lax.broadcasted_iota(jnp.int32, (BLK, t), 0)
               + pl.program_id(0) * BLK)
        perm = (rio == slot_ref[...]).astype(jnp.bfloat16)
        xb = lax.dot_general(perm, xbf_ref[...], (((1,), (0,)), ((), ())),
                             preferred_element_type=jnp.float32
                             ).astype(jnp.bfloat16)
        e_idx = be_ref[pl.program_id(0)]
        w1 = w1_ref[0].astype(jnp.bfloat16)
        h = lax.dot_general(xb, w1, (((1,), (1,)), ((), ())),
                            preferred_element_type=jnp.float32)
        h = jnp.maximum(h + b1_ref[pl.ds(e_idx, 1), :], 0.0)
        w2 = w2_ref[0].astype(jnp.bfloat16)
        y = lax.dot_general(h.astype(jnp.bfloat16), w2, (((1,), (1,)), ((), ())),
                            preferred_element_type=jnp.float32)
        out_ref[...] = y + b2_ref[pl.ds(e_idx, 1), :]


def _grouped_ffn(block_expert, nblk_used, slot_row, x_bf, W1, b1, W2, b2):
    t, d = x_bf.shape
    e, h = b1.shape
    nb = t // BLK + e
    tp = nb * BLK
    grid_spec = pltpu.PrefetchScalarGridSpec(
        num_scalar_prefetch=2,
        grid=(nb,),
        in_specs=[
            pl.BlockSpec((1, t), lambda i, be, nu: (0, 0)),
            pl.BlockSpec((t, d), lambda i, be, nu: (0, 0)),
            pl.BlockSpec((1, h, d), lambda i, be, nu: (be[i], 0, 0)),
            pl.BlockSpec((e, h), lambda i, be, nu: (0, 0)),
            pl.BlockSpec((1, d, h), lambda i, be, nu: (be[i], 0, 0)),
            pl.BlockSpec((e, d), lambda i, be, nu: (0, 0)),
        ],
        out_specs=pl.BlockSpec((BLK, d), lambda i, be, nu: (i, 0)),
    )
    return pl.pallas_call(
        _ffn_body,
        grid_spec=grid_spec,
        out_shape=jax.ShapeDtypeStruct((tp, d), jnp.float32),
    )(block_expert, nblk_used, slot_row, x_bf, W1, b1, W2, b2)


def _sc_gather(table, idx):
    n = idx.shape[0]
    d = table.shape[1]
    rows_per_w = n // NW
    mesh = plsc.VectorSubcoreMesh(core_axis_name="c", subcore_axis_name="s")

    @functools.partial(
        pl.kernel,
        mesh=mesh,
        out_type=jax.ShapeDtypeStruct((n, d), jnp.float32),
        scratch_types=[
            pltpu.VMEM((rows_per_w,), jnp.int32),
            pltpu.VMEM((rows_per_w, d), jnp.float32),
            pltpu.SemaphoreType.DMA,
        ],
    )
    def k(table_hbm, idx_hbm, out_hbm, idx_v, rows_v, sem):
        wid = lax.axis_index("s") * NC + lax.axis_index("c")
        base = wid * rows_per_w
        pltpu.sync_copy(idx_hbm.at[pl.ds(base, rows_per_w)], idx_v)
        pltpu.async_copy(table_hbm.at[idx_v], rows_v, sem).wait()
        pltpu.sync_copy(rows_v, out_hbm.at[pl.ds(base, rows_per_w)])

    return k(table, idx)


def kernel(x, Wg, bg, W1, b1, W2, b2):
    t, d = x.shape
    e = Wg.shape[0]
    nb = t // BLK + e
    tp = nb * BLK

    top_row, slot_row, block_expert, nbu, x_bf = _router(x, Wg, bg)

    ys = _grouped_ffn(block_expert.reshape(nb), nbu.reshape(1),
                      slot_row, x_bf, W1, b1, W2, b2)
    out = _sc_gather(ys, slot_row.reshape(t))
    return (out, top_row.reshape(t, 1))


# speedup vs baseline: 1.2409x; 1.1402x over previous
import functools

import jax
import jax.numpy as jnp
from jax import lax
from jax.experimental import pallas as pl
from jax.experimental.pallas import tpu as pltpu
from jax.experimental.pallas import tpu_sc as plsc

NC, NS = 2, 16
NW = NC * NS
BLK = 256


def _router_body(x_ref, wg_ref, bg_ref, top_ref, slot_ref, be_ref, nbu_ref,
                 xbf_ref):
    xbf_ref[...] = x_ref[...].astype(jnp.bfloat16)
    logits = lax.dot_general(wg_ref[...], x_ref[...], (((1,), (1,)), ((), ())),
                             preferred_element_type=jnp.float32)
    e = logits.shape[0]
    ee_r = lax.broadcasted_iota(jnp.int32, (e, e), 0)
    ee_c = lax.broadcasted_iota(jnp.int32, (e, e), 1)
    bg_col = jnp.sum(jnp.where(ee_r == ee_c, bg_ref[...], 0.0),
                     axis=1, keepdims=True)
    logits = logits + bg_col
    e, t = logits.shape
    nb = t // BLK + e
    maxv = jnp.max(logits, axis=0, keepdims=True)
    ids = lax.broadcasted_iota(jnp.int32, logits.shape, 0)
    picked = jnp.where(logits == maxv, ids, e)
    top = jnp.min(picked, axis=0, keepdims=True)
    top_ref[...] = top

    ohb = ids == top
    ohf = ohb.astype(jnp.bfloat16)
    r_io = lax.broadcasted_iota(jnp.int32, (t, t), 0)
    c_io = lax.broadcasted_iota(jnp.int32, (t, t), 1)
    triu = (r_io <= c_io).astype(jnp.bfloat16)
    csum = lax.dot_general(ohf, triu, (((1,), (0,)), ((), ())),
                           preferred_element_type=jnp.float32)
    rank = jnp.sum(jnp.where(ohb, csum, 0.0), axis=0, keepdims=True) - 1.0
    counts_i = csum[:, t - 1:t].astype(jnp.int32)
    nblk_i = lax.shift_right_logical(counts_i + (BLK - 1), BLK.bit_length() - 1)
    e_r = lax.broadcasted_iota(jnp.int32, (e, e), 0)
    e_c = lax.broadcasted_iota(jnp.int32, (e, e), 1)
    tril_e = (e_r >= e_c).astype(jnp.bfloat16)
    blk_endf = lax.dot_general(tril_e, nblk_i.astype(jnp.bfloat16),
                               (((1,), (0,)), ((), ())),
                               preferred_element_type=jnp.float32)
    blk_end_i = blk_endf.astype(jnp.int32)
    start_tokf = (blk_endf - nblk_i.astype(jnp.float32)) * float(BLK)
    slot_basef = jnp.sum(jnp.where(ohb, start_tokf, 0.0), axis=0, keepdims=True)
    slot_ref[...] = (slot_basef + rank).astype(jnp.int32)
    nbu = blk_end_i[e - 1:e, :]
    nbu_ref[...] = nbu
    b_io = lax.broadcasted_iota(jnp.int32, (e, nb), 1)
    be_raw = jnp.sum((blk_end_i <= b_io).astype(jnp.int32),
                     axis=0, keepdims=True)
    last_e = jnp.sum((blk_end_i <= nbu - 1).astype(jnp.int32),
                     axis=0, keepdims=True)
    be_ref[...] = jnp.where(b_io[0:1, :] < nbu, be_raw, last_e)


def _router(x, Wg, bg):
    t, d = x.shape
    e = Wg.shape[0]
    nb = t // BLK + e
    return pl.pallas_call(
        _router_body,
        out_shape=(
            jax.ShapeDtypeStruct((1, t), jnp.int32),
            jax.ShapeDtypeStruct((1, t), jnp.int32),
            jax.ShapeDtypeStruct((1, nb), jnp.int32),
            jax.ShapeDtypeStruct((1, 1), jnp.int32),
            jax.ShapeDtypeStruct((t, d), jnp.bfloat16),
        ),
    )(x, Wg, bg.reshape(1, -1))


def _ffn_body(be_ref, nblk_ref, slot_ref, xbf_ref, w1_ref, b1_ref, w2_ref,
              b2_ref, out_ref):
    @pl.when(pl.program_id(0) < nblk_ref[0])
    def _():
        t = slot_ref.shape[1]
        rio = (lax.broadcasted_iota(jnp.int32, (BLK, t), 0)
               + pl.program_id(0) * BLK)
        perm = (rio == slot_ref[...]).astype(jnp.bfloat16)
        xb = lax.dot_general(perm, xbf_ref[...], (((1,), (0,)), ((), ())),
                             preferred_element_type=jnp.float32
                             ).astype(jnp.bfloat16)
        e_idx = be_ref[pl.program_id(0)]
        w1 = w1_ref[0].astype(jnp.bfloat16)
        h = lax.dot_general(xb, w1, (((1,), (1,)), ((), ())),
                            preferred_element_type=jnp.float32)
        h = jnp.maximum(h + b1_ref[pl.ds(e_idx, 1), :], 0.0)
        w2 = w2_ref[0].astype(jnp.bfloat16)
        y = lax.dot_general(h.astype(jnp.bfloat16), w2, (((1,), (1,)), ((), ())),
                            preferred_element_type=jnp.float32)
        out_ref[...] = y + b2_ref[pl.ds(e_idx, 1), :]


def _grouped_ffn(block_expert, nblk_used, slot_row, x_bf, W1, b1, W2, b2):
    t, d = x_bf.shape
    e, h = b1.shape
    nb = t // BLK + e
    tp = nb * BLK
    grid_spec = pltpu.PrefetchScalarGridSpec(
        num_scalar_prefetch=2,
        grid=(nb,),
        in_specs=[
            pl.BlockSpec((1, t), lambda i, be, nu: (0, 0)),
            pl.BlockSpec((t, d), lambda i, be, nu: (0, 0)),
            pl.BlockSpec((1, h, d), lambda i, be, nu: (be[i], 0, 0)),
            pl.BlockSpec((e, h), lambda i, be, nu: (0, 0)),
            pl.BlockSpec((1, d, h), lambda i, be, nu: (be[i], 0, 0)),
            pl.BlockSpec((e, d), lambda i, be, nu: (0, 0)),
        ],
        out_specs=pl.BlockSpec((BLK, d), lambda i, be, nu: (i, 0)),
    )
    return pl.pallas_call(
        _ffn_body,
        grid_spec=grid_spec,
        out_shape=jax.ShapeDtypeStruct((tp, d), jnp.float32),
    )(block_expert, nblk_used, slot_row, x_bf, W1, b1, W2, b2)


def _sc_gather(table, idx):
    n = idx.shape[0]
    d = table.shape[1]
    rows_per_w = n // NW
    mesh = plsc.VectorSubcoreMesh(core_axis_name="c", subcore_axis_name="s")

    @functools.partial(
        pl.kernel,
        mesh=mesh,
        out_type=jax.ShapeDtypeStruct((n, d), jnp.float32),
        scratch_types=[
            pltpu.VMEM((rows_per_w,), jnp.int32),
            pltpu.VMEM((rows_per_w, d), jnp.float32),
            pltpu.SemaphoreType.DMA,
        ],
    )
    def k(table_hbm, idx_hbm, out_hbm, idx_v, rows_v, sem):
        wid = lax.axis_index("s") * NC + lax.axis_index("c")
        base = wid * rows_per_w
        pltpu.sync_copy(idx_hbm.at[pl.ds(base, rows_per_w)], idx_v)
        pltpu.async_copy(table_hbm.at[idx_v], rows_v, sem).wait()
        pltpu.sync_copy(rows_v, out_hbm.at[pl.ds(base, rows_per_w)])

    return k(table, idx)


def kernel(x, Wg, bg, W1, b1, W2, b2):
    t, d = x.shape
    e = Wg.shape[0]
    nb = t // BLK + e
    tp = nb * BLK

    top_row, slot_row, block_expert, nbu, x_bf = _router(x, Wg, bg)

    ys = _grouped_ffn(block_expert.reshape(nb), nbu.reshape(1),
                      slot_row, x_bf, W1, b1, W2, b2)
    out = _sc_gather(ys, slot_row.reshape(t))
    return (out, top_row.reshape(t, 1))


# speedup vs baseline: 1.2759x; 1.0282x over previous
import functools

import jax
import jax.numpy as jnp
from jax import lax
from jax.experimental import pallas as pl
from jax.experimental.pallas import tpu as pltpu
from jax.experimental.pallas import tpu_sc as plsc

NC, NS = 2, 16
NW = NC * NS
BLK = 256


def _router_body(x_ref, wg_ref, bg_ref, top_ref, slot_ref, be_ref, nbu_ref,
                 xbf_ref):
    xbf_ref[...] = x_ref[...].astype(jnp.bfloat16)
    logits = lax.dot_general(wg_ref[...], x_ref[...], (((1,), (1,)), ((), ())),
                             preferred_element_type=jnp.float32)
    e = logits.shape[0]
    ee_r = lax.broadcasted_iota(jnp.int32, (e, e), 0)
    ee_c = lax.broadcasted_iota(jnp.int32, (e, e), 1)
    bg_col = jnp.sum(jnp.where(ee_r == ee_c, bg_ref[...], 0.0),
                     axis=1, keepdims=True)
    logits = logits + bg_col
    e, t = logits.shape
    nb = t // BLK + e
    maxv = jnp.max(logits, axis=0, keepdims=True)
    ids = lax.broadcasted_iota(jnp.int32, logits.shape, 0)
    picked = jnp.where(logits == maxv, ids, e)
    top = jnp.min(picked, axis=0, keepdims=True)
    top_ref[...] = top

    ohb = ids == top
    ohf = ohb.astype(jnp.bfloat16)
    r_io = lax.broadcasted_iota(jnp.int32, (t, t), 0)
    c_io = lax.broadcasted_iota(jnp.int32, (t, t), 1)
    triu = (r_io <= c_io).astype(jnp.bfloat16)
    csum = lax.dot_general(ohf, triu, (((1,), (0,)), ((), ())),
                           preferred_element_type=jnp.float32)
    rank = jnp.sum(jnp.where(ohb, csum, 0.0), axis=0, keepdims=True) - 1.0
    counts_i = csum[:, t - 1:t].astype(jnp.int32)
    nblk_i = lax.shift_right_logical(counts_i + (BLK - 1), BLK.bit_length() - 1)
    e_r = lax.broadcasted_iota(jnp.int32, (e, e), 0)
    e_c = lax.broadcasted_iota(jnp.int32, (e, e), 1)
    tril_e = (e_r >= e_c).astype(jnp.bfloat16)
    blk_endf = lax.dot_general(tril_e, nblk_i.astype(jnp.bfloat16),
                               (((1,), (0,)), ((), ())),
                               preferred_element_type=jnp.float32)
    blk_end_i = blk_endf.astype(jnp.int32)
    start_tokf = (blk_endf - nblk_i.astype(jnp.float32)) * float(BLK)
    slot_basef = jnp.sum(jnp.where(ohb, start_tokf, 0.0), axis=0, keepdims=True)
    slot_ref[...] = (slot_basef + rank).astype(jnp.int32)
    nbu = blk_end_i[e - 1:e, :]
    nbu_ref[...] = nbu
    b_io = lax.broadcasted_iota(jnp.int32, (e, nb), 1)
    be_raw = jnp.sum((blk_end_i <= b_io).astype(jnp.int32),
                     axis=0, keepdims=True)
    last_e = jnp.sum((blk_end_i <= nbu - 1).astype(jnp.int32),
                     axis=0, keepdims=True)
    be_ref[...] = jnp.where(b_io[0:1, :] < nbu, be_raw, last_e)


def _router(x, Wg, bg):
    t, d = x.shape
    e = Wg.shape[0]
    nb = t // BLK + e
    return pl.pallas_call(
        _router_body,
        out_shape=(
            jax.ShapeDtypeStruct((1, t), jnp.int32),
            jax.ShapeDtypeStruct((1, t), jnp.int32),
            jax.ShapeDtypeStruct((1, nb), jnp.int32),
            jax.ShapeDtypeStruct((1, 1), jnp.int32),
            jax.ShapeDtypeStruct((t, d), jnp.bfloat16),
        ),
    )(x, Wg, bg.reshape(1, -1))


def _ffn_body(be_ref, nblk_ref, slot_ref, xbf_ref, w1_ref, b1_ref, w2_ref,
              b2_ref, out_ref):
    @pl.when(pl.program_id(0) < nblk_ref[0])
    def _():
        t = slot_ref.shape[1]
        rio = (lax.broadcasted_iota(jnp.int32, (BLK, t), 0)
               + pl.program_id(0) * BLK)
        perm = (rio == slot_ref[...]).astype(jnp.bfloat16)
        xb = lax.dot_general(perm, xbf_ref[...], (((1,), (0,)), ((), ())),
                             preferred_element_type=jnp.float32
                             ).astype(jnp.bfloat16)
        e_idx = be_ref[pl.program_id(0)]
        w1 = w1_ref[0].astype(jnp.bfloat16)
        h = lax.dot_general(xb, w1, (((1,), (1,)), ((), ())),
                            preferred_element_type=jnp.float32)
        h = jnp.maximum(h + b1_ref[pl.ds(e_idx, 1), :], 0.0)
        w2 = w2_ref[0].astype(jnp.bfloat16)
        y = lax.dot_general(h.astype(jnp.bfloat16), w2, (((1,), (1,)), ((), ())),
                            preferred_element_type=jnp.float32)
        out_ref[...] = y + b2_ref[pl.ds(e_idx, 1), :]


def _grouped_ffn(block_expert, nblk_used, slot_row, x_bf, W1, b1, W2, b2):
    t, d = x_bf.shape
    e, h = b1.shape
    nb = t // BLK + e
    tp = nb * BLK
    grid_spec = pltpu.PrefetchScalarGridSpec(
        num_scalar_prefetch=2,
        grid=(nb,),
        in_specs=[
            pl.BlockSpec((1, t), lambda i, be, nu: (0, 0)),
            pl.BlockSpec((t, d), lambda i, be, nu: (0, 0)),
            pl.BlockSpec((1, h, d), lambda i, be, nu: (be[i], 0, 0)),
            pl.BlockSpec((e, h), lambda i, be, nu: (0, 0)),
            pl.BlockSpec((1, d, h), lambda i, be, nu: (be[i], 0, 0)),
            pl.BlockSpec((e, d), lambda i, be, nu: (0, 0)),
        ],
        out_specs=pl.BlockSpec(
            (BLK, d), lambda i, be, nu: (jnp.where(i < nu[0], i, nb - 1), 0)),
    )
    return pl.pallas_call(
        _ffn_body,
        grid_spec=grid_spec,
        out_shape=jax.ShapeDtypeStruct((tp, d), jnp.float32),
    )(block_expert, nblk_used, slot_row, x_bf, W1, b1, W2, b2)


def _sc_gather(table, idx):
    n = idx.shape[0]
    d = table.shape[1]
    rows_per_w = n // NW
    mesh = plsc.VectorSubcoreMesh(core_axis_name="c", subcore_axis_name="s")

    @functools.partial(
        pl.kernel,
        mesh=mesh,
        out_type=jax.ShapeDtypeStruct((n, d), jnp.float32),
        scratch_types=[
            pltpu.VMEM((rows_per_w,), jnp.int32),
            pltpu.VMEM((rows_per_w, d), jnp.float32),
            pltpu.SemaphoreType.DMA,
        ],
    )
    def k(table_hbm, idx_hbm, out_hbm, idx_v, rows_v, sem):
        wid = lax.axis_index("s") * NC + lax.axis_index("c")
        base = wid * rows_per_w
        pltpu.sync_copy(idx_hbm.at[pl.ds(base, rows_per_w)], idx_v)
        pltpu.async_copy(table_hbm.at[idx_v], rows_v, sem).wait()
        pltpu.sync_copy(rows_v, out_hbm.at[pl.ds(base, rows_per_w)])

    return k(table, idx)


def kernel(x, Wg, bg, W1, b1, W2, b2):
    t, d = x.shape
    e = Wg.shape[0]
    nb = t // BLK + e
    tp = nb * BLK

    top_row, slot_row, block_expert, nbu, x_bf = _router(x, Wg, bg)

    ys = _grouped_ffn(block_expert.reshape(nb), nbu.reshape(1),
                      slot_row, x_bf, W1, b1, W2, b2)
    out = _sc_gather(ys, slot_row.reshape(t))
    return (out, top_row.reshape(t, 1))


# speedup vs baseline: 1.5195x; 1.1909x over previous
import functools

import jax
import jax.numpy as jnp
from jax import lax
from jax.experimental import pallas as pl
from jax.experimental.pallas import tpu as pltpu
from jax.experimental.pallas import tpu_sc as plsc

NC, NS = 2, 16
NW = NC * NS
BLK = 256


def _router_body(x_ref, wg_ref, bg_ref, top_ref, slot_ref, be_ref, nbu_ref,
                 xbf_ref):
    xbf_ref[...] = x_ref[...].astype(jnp.bfloat16)
    logits = lax.dot_general(wg_ref[...], x_ref[...], (((1,), (1,)), ((), ())),
                             preferred_element_type=jnp.float32)
    e = logits.shape[0]
    ee_r = lax.broadcasted_iota(jnp.int32, (e, e), 0)
    ee_c = lax.broadcasted_iota(jnp.int32, (e, e), 1)
    bg_col = jnp.sum(jnp.where(ee_r == ee_c, bg_ref[...], 0.0),
                     axis=1, keepdims=True)
    logits = logits + bg_col
    e, t = logits.shape
    nb = t // BLK + e
    maxv = jnp.max(logits, axis=0, keepdims=True)
    ids = lax.broadcasted_iota(jnp.int32, logits.shape, 0)
    picked = jnp.where(logits == maxv, ids, e)
    top = jnp.min(picked, axis=0, keepdims=True)
    top_ref[...] = top

    ohb = ids == top
    ohf = ohb.astype(jnp.bfloat16)
    r_io = lax.broadcasted_iota(jnp.int32, (t, t), 0)
    c_io = lax.broadcasted_iota(jnp.int32, (t, t), 1)
    triu = (r_io <= c_io).astype(jnp.bfloat16)
    csum = lax.dot_general(ohf, triu, (((1,), (0,)), ((), ())),
                           preferred_element_type=jnp.float32)
    rank = jnp.sum(jnp.where(ohb, csum, 0.0), axis=0, keepdims=True) - 1.0
    counts_i = csum[:, t - 1:t].astype(jnp.int32)
    nblk_i = lax.shift_right_logical(counts_i + (BLK - 1), BLK.bit_length() - 1)
    e_r = lax.broadcasted_iota(jnp.int32, (e, e), 0)
    e_c = lax.broadcasted_iota(jnp.int32, (e, e), 1)
    tril_e = (e_r >= e_c).astype(jnp.bfloat16)
    blk_endf = lax.dot_general(tril_e, nblk_i.astype(jnp.bfloat16),
                               (((1,), (0,)), ((), ())),
                               preferred_element_type=jnp.float32)
    blk_end_i = blk_endf.astype(jnp.int32)
    start_tokf = (blk_endf - nblk_i.astype(jnp.float32)) * float(BLK)
    slot_basef = jnp.sum(jnp.where(ohb, start_tokf, 0.0), axis=0, keepdims=True)
    slot_ref[...] = (slot_basef + rank).astype(jnp.int32)
    nbu = blk_end_i[e - 1:e, :]
    nbu_ref[...] = nbu
    b_io = lax.broadcasted_iota(jnp.int32, (e, nb), 1)
    be_raw = jnp.sum((blk_end_i <= b_io).astype(jnp.int32),
                     axis=0, keepdims=True)
    last_e = jnp.sum((blk_end_i <= nbu - 1).astype(jnp.int32),
                     axis=0, keepdims=True)
    be_ref[...] = jnp.where(b_io[0:1, :] < nbu, be_raw, last_e)


def _router(x, Wg, bg):
    t, d = x.shape
    e = Wg.shape[0]
    nb = t // BLK + e
    return pl.pallas_call(
        _router_body,
        out_shape=(
            jax.ShapeDtypeStruct((1, t), jnp.int32),
            jax.ShapeDtypeStruct((1, t), jnp.int32),
            jax.ShapeDtypeStruct((1, nb), jnp.int32),
            jax.ShapeDtypeStruct((1, 1), jnp.int32),
            jax.ShapeDtypeStruct((t, d), jnp.bfloat16),
        ),
    )(x, Wg, bg.reshape(1, -1))


def _ffn_body(be_ref, nblk_ref, slot_ref, xbf_ref, w1_ref, b1_ref, w2_ref,
              b2_ref, out_ref):
    @pl.when(pl.program_id(0) == 0)
    def _():
        out_ref[...] = jnp.zeros_like(out_ref)

    @pl.when(pl.program_id(0) < nblk_ref[0])
    def _():
        t = slot_ref.shape[1]
        rio = (lax.broadcasted_iota(jnp.int32, (BLK, t), 0)
               + pl.program_id(0) * BLK)
        perm = (rio == slot_ref[...]).astype(jnp.bfloat16)
        xb = lax.dot_general(perm, xbf_ref[...], (((1,), (0,)), ((), ())),
                             preferred_element_type=jnp.float32
                             ).astype(jnp.bfloat16)
        e_idx = be_ref[pl.program_id(0)]
        w1 = w1_ref[0].astype(jnp.bfloat16)
        h = lax.dot_general(xb, w1, (((1,), (1,)), ((), ())),
                            preferred_element_type=jnp.float32)
        h = jnp.maximum(h + b1_ref[pl.ds(e_idx, 1), :], 0.0)
        w2 = w2_ref[0].astype(jnp.bfloat16)
        y = lax.dot_general(h.astype(jnp.bfloat16), w2, (((1,), (1,)), ((), ())),
                            preferred_element_type=jnp.float32)
        yb = (y + b2_ref[pl.ds(e_idx, 1), :]).astype(jnp.bfloat16)
        out_ref[...] += lax.dot_general(perm, yb, (((0,), (0,)), ((), ())),
                                        preferred_element_type=jnp.float32)


def _grouped_ffn(block_expert, nblk_used, slot_row, x_bf, W1, b1, W2, b2):
    t, d = x_bf.shape
    e, h = b1.shape
    nb = t // BLK + e
    grid_spec = pltpu.PrefetchScalarGridSpec(
        num_scalar_prefetch=2,
        grid=(nb,),
        in_specs=[
            pl.BlockSpec((1, t), lambda i, be, nu: (0, 0)),
            pl.BlockSpec((t, d), lambda i, be, nu: (0, 0)),
            pl.BlockSpec((1, h, d), lambda i, be, nu: (be[i], 0, 0)),
            pl.BlockSpec((e, h), lambda i, be, nu: (0, 0)),
            pl.BlockSpec((1, d, h), lambda i, be, nu: (be[i], 0, 0)),
            pl.BlockSpec((e, d), lambda i, be, nu: (0, 0)),
        ],
        out_specs=pl.BlockSpec((t, d), lambda i, be, nu: (0, 0)),
    )
    return pl.pallas_call(
        _ffn_body,
        grid_spec=grid_spec,
        out_shape=jax.ShapeDtypeStruct((t, d), jnp.float32),
    )(block_expert, nblk_used, slot_row, x_bf, W1, b1, W2, b2)


def _sc_gather(table, idx):
    n = idx.shape[0]
    d = table.shape[1]
    rows_per_w = n // NW
    mesh = plsc.VectorSubcoreMesh(core_axis_name="c", subcore_axis_name="s")

    @functools.partial(
        pl.kernel,
        mesh=mesh,
        out_type=jax.ShapeDtypeStruct((n, d), jnp.float32),
        scratch_types=[
            pltpu.VMEM((rows_per_w,), jnp.int32),
            pltpu.VMEM((rows_per_w, d), jnp.float32),
            pltpu.SemaphoreType.DMA,
        ],
    )
    def k(table_hbm, idx_hbm, out_hbm, idx_v, rows_v, sem):
        wid = lax.axis_index("s") * NC + lax.axis_index("c")
        base = wid * rows_per_w
        pltpu.sync_copy(idx_hbm.at[pl.ds(base, rows_per_w)], idx_v)
        pltpu.async_copy(table_hbm.at[idx_v], rows_v, sem).wait()
        pltpu.sync_copy(rows_v, out_hbm.at[pl.ds(base, rows_per_w)])

    return k(table, idx)


def kernel(x, Wg, bg, W1, b1, W2, b2):
    t, d = x.shape
    e = Wg.shape[0]
    nb = t // BLK + e
    tp = nb * BLK

    top_row, slot_row, block_expert, nbu, x_bf = _router(x, Wg, bg)

    out = _grouped_ffn(block_expert.reshape(nb), nbu.reshape(1),
                       slot_row, x_bf, W1, b1, W2, b2)
    return (out, top_row.reshape(t, 1))
